# baseline (speedup 1.0000x reference)
import functools

import jax
import jax.numpy as jnp
from jax import lax
from jax.experimental import pallas as pl
from jax.experimental.pallas import tpu as pltpu
from jax.experimental.pallas import tpu_sc as plsc

_INFO = plsc.get_sparse_core_info()
_NC, _NS = _INFO.num_cores, _INFO.num_subcores
_NW = _NC * _NS

_B, _T = 16384, 200
_D = 64
_ROWS_PER_W = _B // _NW
_KB = 4
_IPS = _KB * _T
_G = 80
_NG = _IPS // _G
_N_SLABS = _ROWS_PER_W // _KB


@functools.partial(
    pl.kernel,
    out_type=jax.ShapeDtypeStruct((_B, _T, _D), jnp.float32),
    mesh=plsc.VectorSubcoreMesh(core_axis_name="c", subcore_axis_name="s"),
    scratch_types=[
        pltpu.VMEM((2, _IPS), jnp.int32),
        pltpu.VMEM((2, _IPS, _D), jnp.float32),
        pltpu.SemaphoreType.DMA,
        pltpu.SemaphoreType.DMA,
        pltpu.SemaphoreType.DMA,
        pltpu.SemaphoreType.DMA,
        pltpu.SemaphoreType.DMA,
        pltpu.SemaphoreType.DMA,
    ],
    compiler_params=pltpu.CompilerParams(use_tc_tiling_on_sc=False),
)
def _gather_kernel(idx_hbm, table_hbm, out_hbm, idx_v, rows_v,
                   sem_i0, sem_i1, sem_g0, sem_g1, sem_o0, sem_o1):
    wid = lax.axis_index("s") * _NC + lax.axis_index("c")
    row0 = wid * _ROWS_PER_W
    sem_i, sem_g, sem_o = (sem_i0, sem_i1), (sem_g0, sem_g1), (sem_o0, sem_o1)

    def base(t):
        return row0 + t * _KB

    def idx_copies(t, b):
        return [
            pltpu.make_async_copy(
                idx_hbm.at[base(t) + r],
                idx_v.at[b].at[pl.ds(r * _T, _T)],
                sem_i[b],
            )
            for r in range(_KB)
        ]

    def out_copies(t, b):
        return [
            pltpu.make_async_copy(
                rows_v.at[b].at[pl.ds(r * _T, _T)],
                out_hbm.at[base(t) + r],
                sem_o[b],
            )
            for r in range(_KB)
        ]

    def fire_gathers(b):
        for g in range(_NG):
            pltpu.async_copy(
                table_hbm.at[idx_v.at[b].at[pl.ds(g * _G, _G)]],
                rows_v.at[b].at[pl.ds(g * _G, _G)],
                sem_g[b],
            )

    def drain_gathers(b):
        pltpu.make_async_copy(
            table_hbm.at[pl.ds(0, _IPS)], rows_v.at[b], sem_g[b]
        ).wait()

    def prefetch_idx(t, b):
        for c in idx_copies(t, b):
            c.start()

    def wait_idx(t, b):
        for c in idx_copies(t, b):
            c.wait()

    def writeback(t, b):
        for c in out_copies(t, b):
            c.start()

    def drain_writeback(t, b):
        for c in out_copies(t, b):
            c.wait()

    prefetch_idx(0, 0)
    wait_idx(0, 0)
    fire_gathers(0)
    prefetch_idx(1, 1)
    wait_idx(1, 1)
    fire_gathers(1)
    drain_gathers(0)
    writeback(0, 0)
    prefetch_idx(2, 0)

    def body(i, _):
        s = 2 + 2 * i
        for b in range(2):
            t = s + b
            wait_idx(t, b)
            drain_writeback(t - 2, b)
            fire_gathers(b)
            drain_gathers(1 - b)
            writeback(t - 1, 1 - b)
            tn = jnp.minimum(t + 1, _N_SLABS - 1)
            prefetch_idx(tn, 1 - b)
        return 0

    lax.fori_loop(0, (_N_SLABS - 2) // 2, body, 0, unroll=False)

    last = _N_SLABS - 1
    wait_idx(last, 0)
    drain_gathers(1)
    writeback(last, 1)
    drain_writeback(last - 1, 0)
    drain_writeback(last, 1)


def kernel(data, tvectors):
    return _gather_kernel(data.astype(jnp.int32), tvectors)


# speedup vs baseline: 1.6487x; 1.6487x over previous
import functools

import jax
import jax.numpy as jnp
from jax import lax
from jax.experimental import pallas as pl
from jax.experimental.pallas import tpu as pltpu
from jax.experimental.pallas import tpu_sc as plsc

_INFO = plsc.get_sparse_core_info()
_NC, _NS = _INFO.num_cores, _INFO.num_subcores
_NW = _NC * _NS

_B, _T = 16384, 200
_D = 64
_ROWS_PER_W = _B // _NW
_KB = 4
_IPS = _KB * _T
_G = 80
_NG = _IPS // _G
_N_SLABS = _ROWS_PER_W // _KB


@functools.partial(
    pl.kernel,
    out_type=jax.ShapeDtypeStruct((_B * _T, 2 * _D), jnp.float32),
    mesh=plsc.VectorSubcoreMesh(core_axis_name="c", subcore_axis_name="s"),
    scratch_types=[
        pltpu.VMEM((2, _IPS), jnp.int32),
        pltpu.VMEM((2, _IPS, _D), jnp.float32),
        pltpu.SemaphoreType.DMA,
        pltpu.SemaphoreType.DMA,
        pltpu.SemaphoreType.DMA,
        pltpu.SemaphoreType.DMA,
        pltpu.SemaphoreType.DMA,
        pltpu.SemaphoreType.DMA,
    ],
    compiler_params=pltpu.CompilerParams(use_tc_tiling_on_sc=False),
)
def _gather_kernel(idx_hbm, table_hbm, out_hbm, idx_v, rows_v,
                   sem_i0, sem_i1, sem_g0, sem_g1, sem_o0, sem_o1):
    wid = lax.axis_index("s") * _NC + lax.axis_index("c")
    row0 = wid * _ROWS_PER_W
    sem_i, sem_g, sem_o = (sem_i0, sem_i1), (sem_g0, sem_g1), (sem_o0, sem_o1)

    def base(t):
        return row0 + t * _KB

    def idx_copies(t, b):
        return [
            pltpu.make_async_copy(
                idx_hbm.at[base(t) + r],
                idx_v.at[b].at[pl.ds(r * _T, _T)],
                sem_i[b],
            )
            for r in range(_KB)
        ]

    def out_copies(t, b):
        return [
            pltpu.make_async_copy(
                rows_v.at[b].at[pl.ds(r * _T, _T)],
                out_hbm.at[pl.ds((base(t) + r) * _T, _T), pl.ds(0, _D)],
                sem_o[b],
            )
            for r in range(_KB)
        ]

    def fire_gathers(b):
        for g in range(_NG):
            pltpu.async_copy(
                table_hbm.at[idx_v.at[b].at[pl.ds(g * _G, _G)]],
                rows_v.at[b].at[pl.ds(g * _G, _G)],
                sem_g[b],
            )

    def drain_gathers(b):
        pltpu.make_async_copy(
            table_hbm.at[pl.ds(0, _IPS)], rows_v.at[b], sem_g[b]
        ).wait()

    def prefetch_idx(t, b):
        for c in idx_copies(t, b):
            c.start()

    def wait_idx(t, b):
        for c in idx_copies(t, b):
            c.wait()

    def writeback(t, b):
        for c in out_copies(t, b):
            c.start()

    def drain_writeback(t, b):
        for c in out_copies(t, b):
            c.wait()

    prefetch_idx(0, 0)
    wait_idx(0, 0)
    fire_gathers(0)
    prefetch_idx(1, 1)
    wait_idx(1, 1)
    fire_gathers(1)
    drain_gathers(0)
    writeback(0, 0)
    prefetch_idx(2, 0)

    def body(i, _):
        s = 2 + 2 * i
        for b in range(2):
            t = s + b
            wait_idx(t, b)
            drain_writeback(t - 2, b)
            fire_gathers(b)
            drain_gathers(1 - b)
            writeback(t - 1, 1 - b)
            tn = jnp.minimum(t + 1, _N_SLABS - 1)
            prefetch_idx(tn, 1 - b)
        return 0

    lax.fori_loop(0, (_N_SLABS - 2) // 2, body, 0, unroll=False)

    last = _N_SLABS - 1
    wait_idx(last, 0)
    drain_gathers(1)
    writeback(last, 1)
    drain_writeback(last - 1, 0)
    drain_writeback(last, 1)


def kernel(data, tvectors):
    out = _gather_kernel(data.astype(jnp.int32), tvectors)
    return out.reshape(_B, _T, 2 * _D)[:, :, :_D]


# speedup vs baseline: 1.7038x; 1.0334x over previous
import functools

import jax
import jax.numpy as jnp
from jax import lax
from jax.experimental import pallas as pl
from jax.experimental.pallas import tpu as pltpu
from jax.experimental.pallas import tpu_sc as plsc

_INFO = plsc.get_sparse_core_info()
_NC, _NS = _INFO.num_cores, _INFO.num_subcores
_NW = _NC * _NS

_B, _T = 16384, 200
_D = 64
_V = 1000000
_ROWS_PER_W = _B // _NW
_KB = 4
_IPS = _KB * _T
_G = 80
_NG = _IPS // _G
_N_SLABS = _ROWS_PER_W // _KB


@functools.partial(
    pl.kernel,
    out_type=jax.ShapeDtypeStruct((_B * _T, 2 * _D), jnp.float32),
    mesh=plsc.VectorSubcoreMesh(core_axis_name="c", subcore_axis_name="s"),
    scratch_types=[
        pltpu.VMEM((2, _IPS), jnp.int32),
        pltpu.VMEM((2, _IPS, _D), jnp.float32),
        pltpu.SemaphoreType.DMA,
        pltpu.SemaphoreType.DMA,
        pltpu.SemaphoreType.DMA,
        pltpu.SemaphoreType.DMA,
        pltpu.SemaphoreType.DMA,
        pltpu.SemaphoreType.DMA,
    ],
    compiler_params=pltpu.CompilerParams(use_tc_tiling_on_sc=False),
)
def _gather_kernel(idx_hbm, table_hbm, out_hbm, idx_v, rows_v,
                   sem_i0, sem_i1, sem_g0, sem_g1, sem_o0, sem_o1):
    wid = lax.axis_index("s") * _NC + lax.axis_index("c")
    row0 = wid * _ROWS_PER_W
    sem_i, sem_g, sem_o = (sem_i0, sem_i1), (sem_g0, sem_g1), (sem_o0, sem_o1)

    def base(t):
        return row0 + t * _KB

    def idx_copies(t, b):
        return [
            pltpu.make_async_copy(
                idx_hbm.at[base(t) + r],
                idx_v.at[b].at[pl.ds(r * _T, _T)],
                sem_i[b],
            )
            for r in range(_KB)
        ]

    def out_copies(t, b):
        return [
            pltpu.make_async_copy(
                rows_v.at[b].at[pl.ds(r * _T, _T)],
                out_hbm.at[pl.ds((base(t) + r) * _T, _T), pl.ds(0, _D)],
                sem_o[b],
            )
            for r in range(_KB)
        ]

    def fire_gathers(b):
        for g in range(_NG):
            pltpu.async_copy(
                table_hbm.at[idx_v.at[b].at[pl.ds(g * _G, _G)]],
                rows_v.at[b].at[pl.ds(g * _G, _G)],
                sem_g[b],
            )

    def drain_gathers(b):
        pltpu.make_async_copy(
            table_hbm.at[pl.ds(0, _IPS)], rows_v.at[b], sem_g[b]
        ).wait()

    def prefetch_idx(t, b):
        for c in idx_copies(t, b):
            c.start()

    def wait_idx(t, b):
        for c in idx_copies(t, b):
            c.wait()

    def writeback(t, b):
        for c in out_copies(t, b):
            c.start()

    def drain_writeback(t, b):
        for c in out_copies(t, b):
            c.wait()

    prefetch_idx(0, 0)
    wait_idx(0, 0)
    fire_gathers(0)
    prefetch_idx(1, 1)
    wait_idx(1, 1)
    fire_gathers(1)
    drain_gathers(0)
    writeback(0, 0)
    prefetch_idx(2, 0)

    def body(i, _):
        s = 2 + 2 * i
        for b in range(2):
            t = s + b
            wait_idx(t, b)
            drain_writeback(t - 2, b)
            fire_gathers(b)
            drain_gathers(1 - b)
            writeback(t - 1, 1 - b)
            tn = jnp.minimum(t + 1, _N_SLABS - 1)
            prefetch_idx(tn, 1 - b)
        return 0

    lax.fori_loop(0, (_N_SLABS - 2) // 2, body, 0, unroll=False)

    last = _N_SLABS - 1
    wait_idx(last, 0)
    drain_gathers(1)
    writeback(last, 1)
    drain_writeback(last - 1, 0)
    drain_writeback(last, 1)


def kernel(data, tvectors):
    tvp = jnp.pad(tvectors, ((0, 0), (0, _D))).reshape(2 * _V, _D)
    out = _gather_kernel(data.astype(jnp.int32) * 2, tvp)
    return out.reshape(_B, _T, 2 * _D)[:, :, :_D]


# speedup vs baseline: 1.7041x; 1.0002x over previous
import functools

import jax
import jax.numpy as jnp
from jax import lax
from jax.experimental import pallas as pl
from jax.experimental.pallas import tpu as pltpu
from jax.experimental.pallas import tpu_sc as plsc

_INFO = plsc.get_sparse_core_info()
_NC, _NS = _INFO.num_cores, _INFO.num_subcores
_NW = _NC * _NS

_B, _T = 16384, 200
_D = 64
_V = 1000000
_ROWS_PER_W = _B // _NW
_KB = 4
_IPS = _KB * _T
_CHUNKS = [(0, 128), (128, 128), (256, 128), (384, 128),
           (512, 128), (640, 128), (768, 32)]
_N_SLABS = _ROWS_PER_W // _KB


@functools.partial(
    pl.kernel,
    out_type=jax.ShapeDtypeStruct((_B * _T, 2 * _D), jnp.float32),
    mesh=plsc.VectorSubcoreMesh(core_axis_name="c", subcore_axis_name="s"),
    scratch_types=[
        pltpu.VMEM((2, _IPS), jnp.int32),
        pltpu.VMEM((2, _IPS, _D), jnp.float32),
        pltpu.SemaphoreType.DMA,
        pltpu.SemaphoreType.DMA,
        pltpu.SemaphoreType.DMA,
        pltpu.SemaphoreType.DMA,
        pltpu.SemaphoreType.DMA,
        pltpu.SemaphoreType.DMA,
    ],
    compiler_params=pltpu.CompilerParams(use_tc_tiling_on_sc=False),
)
def _gather_kernel(idx_hbm, table_hbm, out_hbm, idx_v, rows_v,
                   sem_i0, sem_i1, sem_g0, sem_g1, sem_o0, sem_o1):
    wid = lax.axis_index("s") * _NC + lax.axis_index("c")
    row0 = wid * _ROWS_PER_W
    sem_i, sem_g, sem_o = (sem_i0, sem_i1), (sem_g0, sem_g1), (sem_o0, sem_o1)

    def base(t):
        return row0 + t * _KB

    def idx_copies(t, b):
        return [
            pltpu.make_async_copy(
                idx_hbm.at[base(t) + r],
                idx_v.at[b].at[pl.ds(r * _T, _T)],
                sem_i[b],
            )
            for r in range(_KB)
        ]

    def out_copies(t, b):
        return [
            pltpu.make_async_copy(
                rows_v.at[b].at[pl.ds(r * _T, _T)],
                out_hbm.at[pl.ds((base(t) + r) * _T, _T), pl.ds(0, _D)],
                sem_o[b],
            )
            for r in range(_KB)
        ]

    def fire_gathers(b):
        for off, n in _CHUNKS:
            pltpu.async_copy(
                table_hbm.at[idx_v.at[b].at[pl.ds(off, n)]],
                rows_v.at[b].at[pl.ds(off, n)],
                sem_g[b],
            )

    def drain_gathers(b):
        pltpu.make_async_copy(
            table_hbm.at[pl.ds(0, _IPS)], rows_v.at[b], sem_g[b]
        ).wait()

    def prefetch_idx(t, b):
        for c in idx_copies(t, b):
            c.start()

    def wait_idx(t, b):
        for c in idx_copies(t, b):
            c.wait()

    def writeback(t, b):
        for c in out_copies(t, b):
            c.start()

    def drain_writeback(t, b):
        for c in out_copies(t, b):
            c.wait()

    prefetch_idx(0, 0)
    wait_idx(0, 0)
    fire_gathers(0)
    prefetch_idx(1, 1)
    wait_idx(1, 1)
    fire_gathers(1)
    drain_gathers(0)
    writeback(0, 0)
    prefetch_idx(2, 0)

    def body(i, _):
        s = 2 + 2 * i
        for b in range(2):
            t = s + b
            wait_idx(t, b)
            drain_writeback(t - 2, b)
            fire_gathers(b)
            drain_gathers(1 - b)
            writeback(t - 1, 1 - b)
            tn = jnp.minimum(t + 1, _N_SLABS - 1)
            prefetch_idx(tn, 1 - b)
        return 0

    lax.fori_loop(0, (_N_SLABS - 2) // 2, body, 0, unroll=False)

    last = _N_SLABS - 1
    wait_idx(last, 0)
    drain_gathers(1)
    writeback(last, 1)
    drain_writeback(last - 1, 0)
    drain_writeback(last, 1)


def kernel(data, tvectors):
    tvp = jnp.pad(tvectors, ((0, 0), (0, _D))).reshape(2 * _V, _D)
    out = _gather_kernel(data.astype(jnp.int32) * 2, tvp)
    return out.reshape(_B, _T, 2 * _D)[:, :, :_D]


# speedup vs baseline: 1.7079x; 1.0022x over previous
import functools

import jax
import jax.numpy as jnp
from jax import lax
from jax.experimental import pallas as pl
from jax.experimental.pallas import tpu as pltpu
from jax.experimental.pallas import tpu_sc as plsc

_INFO = plsc.get_sparse_core_info()
_NC, _NS = _INFO.num_cores, _INFO.num_subcores
_NW = _NC * _NS

_B, _T = 16384, 200
_D = 64
_V = 1000000
_ROWS_PER_W = _B // _NW
_KB = 4
_IPS = _KB * _T
_CHUNKS = [(0, 128), (128, 128), (256, 128), (384, 128),
           (512, 128), (640, 128), (768, 32)]
_N_SLABS = _ROWS_PER_W // _KB


@functools.partial(
    pl.kernel,
    out_type=jax.ShapeDtypeStruct((_B * _T, 2 * _D), jnp.float32),
    mesh=plsc.VectorSubcoreMesh(core_axis_name="c", subcore_axis_name="s"),
    scratch_types=[
        pltpu.VMEM((2, _IPS), jnp.int32),
        pltpu.VMEM((2, _IPS, _D), jnp.float32),
        pltpu.SemaphoreType.DMA,
        pltpu.SemaphoreType.DMA,
        pltpu.SemaphoreType.DMA,
        pltpu.SemaphoreType.DMA,
        pltpu.SemaphoreType.DMA,
        pltpu.SemaphoreType.DMA,
    ],
    compiler_params=pltpu.CompilerParams(use_tc_tiling_on_sc=False),
)
def _gather_kernel(idx_hbm, table_hbm, out_hbm, idx_v, rows_v,
                   sem_i0, sem_i1, sem_g0, sem_g1, sem_o0, sem_o1):
    wid = lax.axis_index("s") * _NC + lax.axis_index("c")
    row0 = wid * _ROWS_PER_W
    sem_i, sem_g, sem_o = (sem_i0, sem_i1), (sem_g0, sem_g1), (sem_o0, sem_o1)

    def base(t):
        return row0 + t * _KB

    def idx_copies(t, b):
        return [
            pltpu.make_async_copy(
                idx_hbm.at[base(t) + r],
                idx_v.at[b].at[pl.ds(r * _T, _T)],
                sem_i[b],
            )
            for r in range(_KB)
        ]

    def out_copies(t, b):
        return [
            pltpu.make_async_copy(
                rows_v.at[b].at[pl.ds(r * _T, _T)],
                out_hbm.at[pl.ds((base(t) + r) * _T, _T), pl.ds(0, _D)],
                sem_o[b],
            )
            for r in range(_KB)
        ]

    def fire_gathers(b):
        for off, n in _CHUNKS:
            pltpu.async_copy(
                table_hbm.at[idx_v.at[b].at[pl.ds(off, n)]],
                rows_v.at[b].at[pl.ds(off, n)],
                sem_g[b],
            )

    def drain_gathers(b):
        pltpu.make_async_copy(
            table_hbm.at[pl.ds(0, _IPS)], rows_v.at[b], sem_g[b]
        ).wait()

    def prefetch_idx(t, b):
        for c in idx_copies(t, b):
            c.start()

    def wait_idx(t, b):
        for c in idx_copies(t, b):
            c.wait()

    def writeback(t, b):
        for c in out_copies(t, b):
            c.start()

    def drain_writeback(t, b):
        for c in out_copies(t, b):
            c.wait()

    prefetch_idx(0, 0)
    wait_idx(0, 0)
    fire_gathers(0)
    prefetch_idx(1, 1)
    wait_idx(1, 1)
    fire_gathers(1)
    drain_gathers(0)
    writeback(0, 0)
    prefetch_idx(2, 0)

    def body(i, _):
        s = 2 + 2 * i
        for b in range(2):
            t = s + b
            wait_idx(t, b)
            drain_writeback(t - 2, b)
            fire_gathers(b)
            drain_gathers(1 - b)
            writeback(t - 1, 1 - b)
            tn = jnp.minimum(t + 1, _N_SLABS - 1)
            prefetch_idx(tn, 1 - b)
        return 0

    lax.fori_loop(0, (_N_SLABS - 2) // 2, body, 0, unroll=False)

    last = _N_SLABS - 1
    wait_idx(last, 0)
    drain_gathers(1)
    writeback(last, 1)
    drain_writeback(last - 1, 0)
    drain_writeback(last, 1)


def kernel(data, tvectors):
    tvp = jnp.concatenate(
        [tvectors, jnp.zeros((_V, _D), jnp.float32)], axis=1
    ).reshape(2 * _V, _D)
    out = _gather_kernel(data.astype(jnp.int32) * 2, tvp)
    return out.reshape(_B, _T, 2 * _D)[:, :, :_D]
